# Initial kernel scaffold; baseline (speedup 1.0000x reference)
import jax
import jax.numpy as jnp
from jax.experimental import pallas as pl


def kernel(x, edge_index, edge_attr, batch, W1, b1, W2, b2, W3, b3):
    raise NotImplementedError("write your pallas kernel here")


# speedup vs baseline: 17.8731x; 17.8731x over previous
import jax
import jax.numpy as jnp
from jax import lax
from jax.experimental import pallas as pl
from jax.experimental.pallas import tpu as pltpu
from jax.experimental.pallas import tpu_sc as plsc

N = 10000
NP = 10240
E = 160000
DIN = 256
HD = 128
H = 512
OUT = 40
NG = 64
NSUB = 16
EPS = E // NSUB
CH = 80
NCH = EPS // CH
RPS = NP // NSUB
CW = NP * NG
CPS = CW // NSUB
BLK = 1000
GRID = N // BLK

_SC_PARAMS = pltpu.CompilerParams(needs_layout_passes=False)


def _sc_agg_body(x2_hbm, src_hbm, dst_hbm,
                 agg0_hbm, agg1_hbm,
                 src_v, dst_v, rows_v, agg_sh):
    c = lax.axis_index("c")
    s = lax.axis_index("s")

    pltpu.sync_copy(src_hbm.at[s], src_v)
    pltpu.sync_copy(dst_hbm.at[s], dst_v)
    pltpu.sync_copy(x2_hbm.at[pl.ds(c * NP + s * RPS, RPS), :],
                    agg_sh.at[pl.ds(s * RPS, RPS), :])

    @pl.when(c == 1)
    def _():
        def build(r, carry):
            for k in range(CH // 16):
                sl = pl.ds(k * 16, 16)
                src_v[r, sl] = src_v[r, sl] + NP
            return carry
        lax.fori_loop(0, NCH, build, 0)

    plsc.subcore_barrier()

    def step(j, carry):
        pltpu.sync_copy(x2_hbm.at[src_v.at[j]], rows_v)
        pltpu.sync_copy(rows_v, agg_sh.at[dst_v.at[j]], add=True)
        return carry

    lax.fori_loop(0, NCH, step, 0)
    plsc.subcore_barrier()

    @pl.when(c == 0)
    def _():
        pltpu.sync_copy(agg_sh.at[pl.ds(s * RPS, RPS), :],
                        agg0_hbm.at[pl.ds(s * RPS, RPS), :])

    @pl.when(c == 1)
    def _():
        pltpu.sync_copy(agg_sh.at[pl.ds(s * RPS, RPS), :],
                        agg1_hbm.at[pl.ds(s * RPS, RPS), :])


_sc_agg = pl.kernel(
    _sc_agg_body,
    mesh=plsc.VectorSubcoreMesh(core_axis_name="c", subcore_axis_name="s"),
    compiler_params=_SC_PARAMS,
    out_type=[jax.ShapeDtypeStruct((NP, HD), jnp.float32),
              jax.ShapeDtypeStruct((NP, HD), jnp.float32)],
    scratch_types=[
        pltpu.VMEM((NCH, CH), jnp.int32),
        pltpu.VMEM((NCH, CH), jnp.int32),
        pltpu.VMEM((CH, HD), jnp.float32),
        pltpu.VMEM_SHARED((NP, HD), jnp.float32),
    ],
)


def _sc_cnt_body(src_hbm, dst_hbm, ea0_hbm, ea1_hbm, ea2_hbm, batch_hbm,
                 zc_hbm, znp_hbm,
                 cnt_hbm, e0_hbm, e1_hbm, e2_hbm,
                 src_v, dst_v, batch_v, ea0_v, ea1_v, ea2_v, ones_v,
                 c_sh, e0_sh, e1_sh, e2_sh):
    c = lax.axis_index("c")
    s = lax.axis_index("s")

    pltpu.sync_copy(src_hbm.at[s], src_v)
    pltpu.sync_copy(dst_hbm.at[s], dst_v)

    @pl.when(c == 0)
    def _():
        pltpu.sync_copy(batch_hbm, batch_v)
        pltpu.sync_copy(zc_hbm.at[pl.ds(s * CPS, CPS)],
                        c_sh.at[pl.ds(s * CPS, CPS)])
        for k in range(CH // 16):
            ones_v[pl.ds(k * 16, 16)] = jnp.ones((16,), jnp.float32)

    @pl.when(c == 1)
    def _():
        pltpu.sync_copy(ea0_hbm.at[pl.ds(s * EPS, EPS)], ea0_v)
        pltpu.sync_copy(ea1_hbm.at[pl.ds(s * EPS, EPS)], ea1_v)
        pltpu.sync_copy(ea2_hbm.at[pl.ds(s * EPS, EPS)], ea2_v)
        pltpu.sync_copy(znp_hbm.at[pl.ds(s * RPS, RPS)],
                        e0_sh.at[pl.ds(s * RPS, RPS)])
        pltpu.sync_copy(znp_hbm.at[pl.ds(s * RPS, RPS)],
                        e1_sh.at[pl.ds(s * RPS, RPS)])
        pltpu.sync_copy(znp_hbm.at[pl.ds(s * RPS, RPS)],
                        e2_sh.at[pl.ds(s * RPS, RPS)])

    @pl.when(c == 0)
    def _():
        def build(r, carry):
            for k in range(CH // 16):
                sl = pl.ds(k * 16, 16)
                d16 = dst_v[r, sl]
                g16 = plsc.load_gather(batch_v, [d16])
                src_v[r, sl] = src_v[r, sl] * NG + g16
            return carry
        lax.fori_loop(0, NCH, build, 0)

    plsc.subcore_barrier()

    def step(j, carry):
        @pl.when(c == 0)
        def _():
            pltpu.sync_copy(ones_v, c_sh.at[src_v.at[j]], add=True)

        @pl.when(c == 1)
        def _():
            pltpu.sync_copy(ea0_v.at[pl.ds(j * CH, CH)],
                            e0_sh.at[dst_v.at[j]], add=True)
            pltpu.sync_copy(ea1_v.at[pl.ds(j * CH, CH)],
                            e1_sh.at[dst_v.at[j]], add=True)
            pltpu.sync_copy(ea2_v.at[pl.ds(j * CH, CH)],
                            e2_sh.at[dst_v.at[j]], add=True)
        return carry

    lax.fori_loop(0, NCH, step, 0)
    plsc.subcore_barrier()

    @pl.when(c == 0)
    def _():
        pltpu.sync_copy(c_sh.at[pl.ds(s * CPS, CPS)],
                        cnt_hbm.at[pl.ds(s * CPS, CPS)])

    @pl.when(c == 1)
    def _():
        pltpu.sync_copy(e0_sh.at[pl.ds(s * RPS, RPS)],
                        e0_hbm.at[pl.ds(s * RPS, RPS)])
        pltpu.sync_copy(e1_sh.at[pl.ds(s * RPS, RPS)],
                        e1_hbm.at[pl.ds(s * RPS, RPS)])
        pltpu.sync_copy(e2_sh.at[pl.ds(s * RPS, RPS)],
                        e2_hbm.at[pl.ds(s * RPS, RPS)])


_sc_cnt = pl.kernel(
    _sc_cnt_body,
    mesh=plsc.VectorSubcoreMesh(core_axis_name="c", subcore_axis_name="s"),
    compiler_params=_SC_PARAMS,
    out_type=[jax.ShapeDtypeStruct((CW,), jnp.float32),
              jax.ShapeDtypeStruct((NP,), jnp.float32),
              jax.ShapeDtypeStruct((NP,), jnp.float32),
              jax.ShapeDtypeStruct((NP,), jnp.float32)],
    scratch_types=[
        pltpu.VMEM((NCH, CH), jnp.int32),
        pltpu.VMEM((NCH, CH), jnp.int32),
        pltpu.VMEM((N,), jnp.int32),
        pltpu.VMEM((EPS,), jnp.float32),
        pltpu.VMEM((EPS,), jnp.float32),
        pltpu.VMEM((EPS,), jnp.float32),
        pltpu.VMEM((CH,), jnp.float32),
        pltpu.VMEM_SHARED((CW,), jnp.float32),
        pltpu.VMEM_SHARED((NP,), jnp.float32),
        pltpu.VMEM_SHARED((NP,), jnp.float32),
        pltpu.VMEM_SHARED((NP,), jnp.float32),
    ],
)


def _tc_body(agg0, agg1, cmat, eas, batch, w1a, w1b, w1e, b1, w2x, w2e, b2,
             w3, b3, out, p_acc, q_acc):
    i = pl.program_id(0)

    @pl.when(i == 0)
    def _():
        p_acc[...] = jnp.zeros_like(p_acc)
        q_acc[...] = jnp.zeros_like(q_acc)

    h = (jnp.dot(agg0[...], w1a[...], preferred_element_type=jnp.float32)
         + jnp.dot(agg1[...], w1b[...], preferred_element_type=jnp.float32)
         + jnp.dot(eas[...], w1e[...], preferred_element_type=jnp.float32)
         + b1[...])
    h = jnp.maximum(h, 0.0)
    gids = lax.broadcasted_iota(jnp.int32, (1, NG), 1)
    oneh = (batch[...] == gids).astype(jnp.float32)
    m = oneh + cmat[...]
    p_acc[...] += lax.dot_general(m, h, (((0,), (0,)), ((), ())),
                                  preferred_element_type=jnp.float32)
    eaone = jnp.concatenate(
        [eas[...], jnp.ones((BLK, 1), jnp.float32),
         jnp.zeros((BLK, 3), jnp.float32)], axis=1)
    q_acc[...] += lax.dot_general(oneh, eaone, (((0,), (0,)), ((), ())),
                                  preferred_element_type=jnp.float32)

    @pl.when(i == GRID - 1)
    def _():
        q = q_acc[...]
        inv = 1.0 / jnp.maximum(q[:, 4:5], 1.0)
        pooled = (jnp.dot(p_acc[...] * inv, w2x[...],
                          preferred_element_type=jnp.float32)
                  + jnp.dot(q[:, :4] * inv, w2e[...],
                            preferred_element_type=jnp.float32)
                  + b2[...])
        logits = jnp.dot(pooled, w3[...],
                         preferred_element_type=jnp.float32) + b3[...]
        mx = jnp.max(logits, axis=1, keepdims=True)
        lse = jnp.log(jnp.sum(jnp.exp(logits - mx), axis=1, keepdims=True))
        out[...] = logits - mx - lse


_tc_head = pl.pallas_call(
    _tc_body,
    grid=(GRID,),
    in_specs=[
        pl.BlockSpec((BLK, HD), lambda i: (i, 0)),
        pl.BlockSpec((BLK, HD), lambda i: (i, 0)),
        pl.BlockSpec((BLK, NG), lambda i: (i, 0)),
        pl.BlockSpec((BLK, 4), lambda i: (i, 0)),
        pl.BlockSpec((BLK, 1), lambda i: (i, 0)),
        pl.BlockSpec((HD, H), lambda i: (0, 0)),
        pl.BlockSpec((HD, H), lambda i: (0, 0)),
        pl.BlockSpec((4, H), lambda i: (0, 0)),
        pl.BlockSpec((1, H), lambda i: (0, 0)),
        pl.BlockSpec((H, H), lambda i: (0, 0)),
        pl.BlockSpec((4, H), lambda i: (0, 0)),
        pl.BlockSpec((1, H), lambda i: (0, 0)),
        pl.BlockSpec((H, OUT), lambda i: (0, 0)),
        pl.BlockSpec((1, OUT), lambda i: (0, 0)),
    ],
    out_specs=pl.BlockSpec((NG, OUT), lambda i: (0, 0)),
    out_shape=jax.ShapeDtypeStruct((NG, OUT), jnp.float32),
    scratch_shapes=[pltpu.VMEM((NG, H), jnp.float32),
                    pltpu.VMEM((NG, 8), jnp.float32)],
)


def kernel(x, edge_index, edge_attr, batch, W1, b1, W2, b2, W3, b3):
    zpad = jnp.zeros((NP - N, HD), jnp.float32)
    x2 = jnp.concatenate([x[:, :HD], zpad, x[:, HD:], zpad], axis=0)
    srcr = edge_index[0].reshape(NSUB, NCH, CH)
    dstr = edge_index[1].reshape(NSUB, NCH, CH)
    zc = jnp.zeros((CW,), jnp.float32)
    znp = jnp.zeros((NP,), jnp.float32)
    agg0, agg1 = _sc_agg(x2, srcr, dstr)
    cflat, e0, e1, e2 = _sc_cnt(srcr, dstr, edge_attr[:, 0], edge_attr[:, 1],
                                edge_attr[:, 2], batch, zc, znp)
    cmat = cflat.reshape(NP, NG)
    eas = jnp.stack([e0, e1, e2, znp], axis=1)
    batch2 = batch.reshape(N, 1)
    w1a = W1[:, :HD].T
    w1b = W1[:, HD:DIN].T
    w1e = jnp.pad(W1[:, DIN:].T, ((0, 1), (0, 0)))
    b1r = b1.reshape(1, H)
    w2x = W2[:, :H].T
    w2e = jnp.pad(W2[:, H:].T, ((0, 1), (0, 0)))
    b2r = b2.reshape(1, H)
    w3t = W3.T
    b3r = b3.reshape(1, OUT)
    return _tc_head(agg0, agg1, cmat, eas, batch2, w1a, w1b, w1e, b1r,
                    w2x, w2e, b2r, w3t, b3r)


# speedup vs baseline: 23.2298x; 1.2997x over previous
import jax
import jax.numpy as jnp
from jax import lax
from jax.experimental import pallas as pl
from jax.experimental.pallas import tpu as pltpu
from jax.experimental.pallas import tpu_sc as plsc

N = 10000
NP = 10240
E = 160000
DIN = 256
HD = 128
H = 512
OUT = 40
NG = 64
NSUB = 16
EPS = E // NSUB
CH = 80
NCH = EPS // CH
RPS = NP // NSUB
CW = NP * NG
CPS = CW // NSUB
BLK = 1024
GRID = NP // BLK

_SC_PARAMS = pltpu.CompilerParams(needs_layout_passes=False)


def _sc_agg_body(x2_hbm, srcf_hbm, dst_hbm, agg_hbm,
                 src_v, dst_v, rows0_v, rows1_v, agg_sh, gsem0, gsem1):
    c = lax.axis_index("c")
    s = lax.axis_index("s")

    pltpu.sync_copy(srcf_hbm.at[s], src_v)
    pltpu.sync_copy(dst_hbm.at[s], dst_v)
    pltpu.sync_copy(x2_hbm.at[pl.ds(c * NP + s * RPS, RPS), :],
                    agg_sh.at[pl.ds(s * RPS, RPS), :])

    @pl.when(c == 1)
    def _():
        def build(r, carry):
            sl = pl.ds(r * 16, 16)
            src_v[sl] = src_v[sl] + NP
            return carry
        lax.fori_loop(0, EPS // 16, build, 0)

    plsc.subcore_barrier()

    def gather(j, buf, sem):
        pltpu.async_copy(x2_hbm.at[src_v.at[pl.ds(j * CH, CH)]], buf, sem)

    def gwait(buf, sem):
        pltpu.make_async_copy(x2_hbm.at[pl.ds(0, CH), :], buf, sem).wait()

    gather(0, rows0_v, gsem0)

    def step(t, carry):
        j0 = 2 * t
        gwait(rows0_v, gsem0)
        gather(j0 + 1, rows1_v, gsem1)
        pltpu.sync_copy(rows0_v, agg_sh.at[dst_v.at[j0]], add=True)
        gwait(rows1_v, gsem1)
        gather(j0 + 2, rows0_v, gsem0)
        pltpu.sync_copy(rows1_v, agg_sh.at[dst_v.at[j0 + 1]], add=True)
        return carry

    lax.fori_loop(0, (NCH - 1) // 2, step, 0)
    gwait(rows0_v, gsem0)
    pltpu.sync_copy(rows0_v, agg_sh.at[dst_v.at[NCH - 1]], add=True)
    plsc.subcore_barrier()

    pltpu.sync_copy(agg_sh.at[pl.ds(s * RPS, RPS), :],
                    agg_hbm.at[pl.ds(c * NP + s * RPS, RPS), :])


_sc_agg = pl.kernel(
    _sc_agg_body,
    mesh=plsc.VectorSubcoreMesh(core_axis_name="c", subcore_axis_name="s"),
    compiler_params=_SC_PARAMS,
    out_type=[jax.ShapeDtypeStruct((2 * NP, HD), jnp.float32)],
    scratch_types=[
        pltpu.VMEM((EPS,), jnp.int32),
        pltpu.VMEM((NCH, CH), jnp.int32),
        pltpu.VMEM((CH, HD), jnp.float32),
        pltpu.VMEM((CH, HD), jnp.float32),
        pltpu.VMEM_SHARED((NP, HD), jnp.float32),
        pltpu.SemaphoreType.DMA,
        pltpu.SemaphoreType.DMA,
    ],
)


def _sc_cnt_body(src_hbm, dst_hbm, ea0_hbm, ea1_hbm, ea2_hbm, batch_hbm,
                 zc_hbm, znp_hbm,
                 cnt_hbm, e0_hbm, e1_hbm, e2_hbm,
                 src_v, dst_v, batch_v, ea0_v, ea1_v, ea2_v, ones_v,
                 c_sh, e0_sh, e1_sh, e2_sh, ssem):
    c = lax.axis_index("c")
    s = lax.axis_index("s")

    pltpu.sync_copy(src_hbm.at[s], src_v)
    pltpu.sync_copy(dst_hbm.at[s], dst_v)

    @pl.when(c == 0)
    def _():
        pltpu.sync_copy(batch_hbm, batch_v)
        pltpu.sync_copy(zc_hbm.at[pl.ds(s * CPS, CPS)],
                        c_sh.at[pl.ds(s * CPS, CPS)])
        for k in range(CH // 16):
            ones_v[pl.ds(k * 16, 16)] = jnp.ones((16,), jnp.float32)

    @pl.when(c == 1)
    def _():
        pltpu.sync_copy(ea0_hbm.at[pl.ds(s * EPS, EPS)], ea0_v)
        pltpu.sync_copy(ea1_hbm.at[pl.ds(s * EPS, EPS)], ea1_v)
        pltpu.sync_copy(ea2_hbm.at[pl.ds(s * EPS, EPS)], ea2_v)
        pltpu.sync_copy(znp_hbm.at[pl.ds(s * RPS, RPS)],
                        e0_sh.at[pl.ds(s * RPS, RPS)])
        pltpu.sync_copy(znp_hbm.at[pl.ds(s * RPS, RPS)],
                        e1_sh.at[pl.ds(s * RPS, RPS)])
        pltpu.sync_copy(znp_hbm.at[pl.ds(s * RPS, RPS)],
                        e2_sh.at[pl.ds(s * RPS, RPS)])

    @pl.when(c == 0)
    def _():
        def build(r, carry):
            for k in range(CH // 16):
                sl = pl.ds(k * 16, 16)
                d16 = dst_v[r, sl]
                g16 = plsc.load_gather(batch_v, [d16])
                src_v[r, sl] = src_v[r, sl] * NG + g16
            return carry
        lax.fori_loop(0, NCH, build, 0)

    plsc.subcore_barrier()

    def step(j, carry):
        @pl.when(c == 0)
        def _():
            pltpu.async_copy(ones_v, c_sh.at[src_v.at[j]], ssem, add=True)

        @pl.when(c == 1)
        def _():
            pltpu.async_copy(ea0_v.at[pl.ds(j * CH, CH)],
                             e0_sh.at[dst_v.at[j]], ssem, add=True)
            pltpu.async_copy(ea1_v.at[pl.ds(j * CH, CH)],
                             e1_sh.at[dst_v.at[j]], ssem, add=True)
            pltpu.async_copy(ea2_v.at[pl.ds(j * CH, CH)],
                             e2_sh.at[dst_v.at[j]], ssem, add=True)
        return carry

    lax.fori_loop(0, NCH, step, 0)

    def drain(j, carry):
        @pl.when(c == 0)
        def _():
            pltpu.make_async_copy(zc_hbm.at[pl.ds(0, CH)], ones_v, ssem).wait()

        @pl.when(c == 1)
        def _():
            pltpu.make_async_copy(znp_hbm.at[pl.ds(0, CH)],
                                  ea0_v.at[pl.ds(0, CH)], ssem).wait()
            pltpu.make_async_copy(znp_hbm.at[pl.ds(0, CH)],
                                  ea1_v.at[pl.ds(0, CH)], ssem).wait()
            pltpu.make_async_copy(znp_hbm.at[pl.ds(0, CH)],
                                  ea2_v.at[pl.ds(0, CH)], ssem).wait()
        return carry

    lax.fori_loop(0, NCH, drain, 0)
    plsc.subcore_barrier()

    @pl.when(c == 0)
    def _():
        pltpu.sync_copy(c_sh.at[pl.ds(s * CPS, CPS)],
                        cnt_hbm.at[pl.ds(s * CPS, CPS)])

    @pl.when(c == 1)
    def _():
        pltpu.sync_copy(e0_sh.at[pl.ds(s * RPS, RPS)],
                        e0_hbm.at[pl.ds(s * RPS, RPS)])
        pltpu.sync_copy(e1_sh.at[pl.ds(s * RPS, RPS)],
                        e1_hbm.at[pl.ds(s * RPS, RPS)])
        pltpu.sync_copy(e2_sh.at[pl.ds(s * RPS, RPS)],
                        e2_hbm.at[pl.ds(s * RPS, RPS)])


_sc_cnt = pl.kernel(
    _sc_cnt_body,
    mesh=plsc.VectorSubcoreMesh(core_axis_name="c", subcore_axis_name="s"),
    compiler_params=_SC_PARAMS,
    out_type=[jax.ShapeDtypeStruct((CW,), jnp.float32),
              jax.ShapeDtypeStruct((NP,), jnp.float32),
              jax.ShapeDtypeStruct((NP,), jnp.float32),
              jax.ShapeDtypeStruct((NP,), jnp.float32)],
    scratch_types=[
        pltpu.VMEM((NCH, CH), jnp.int32),
        pltpu.VMEM((NCH, CH), jnp.int32),
        pltpu.VMEM((N,), jnp.int32),
        pltpu.VMEM((EPS,), jnp.float32),
        pltpu.VMEM((EPS,), jnp.float32),
        pltpu.VMEM((EPS,), jnp.float32),
        pltpu.VMEM((CH,), jnp.float32),
        pltpu.VMEM_SHARED((CW,), jnp.float32),
        pltpu.VMEM_SHARED((NP,), jnp.float32),
        pltpu.VMEM_SHARED((NP,), jnp.float32),
        pltpu.VMEM_SHARED((NP,), jnp.float32),
        pltpu.SemaphoreType.DMA,
    ],
)


def _tc_body(agg0, agg1, cmat, eas, batch, w1a, w1b, w1e, b1, w2x, w2e, b2,
             w3, b3, out, p_acc, q_acc):
    i = pl.program_id(0)

    @pl.when(i == 0)
    def _():
        p_acc[...] = jnp.zeros_like(p_acc)
        q_acc[...] = jnp.zeros_like(q_acc)

    h = (jnp.dot(agg0[...], w1a[...], preferred_element_type=jnp.float32)
         + jnp.dot(agg1[...], w1b[...], preferred_element_type=jnp.float32)
         + jnp.dot(eas[...], w1e[...], preferred_element_type=jnp.float32)
         + b1[...])
    h = jnp.maximum(h, 0.0)
    gids = lax.broadcasted_iota(jnp.int32, (1, NG), 1)
    oneh = (batch[...] == gids).astype(jnp.float32)
    m = oneh + cmat[...]
    p_acc[...] += lax.dot_general(m, h, (((0,), (0,)), ((), ())),
                                  preferred_element_type=jnp.float32)
    eaone = jnp.concatenate(
        [eas[...], jnp.ones((BLK, 1), jnp.float32),
         jnp.zeros((BLK, 3), jnp.float32)], axis=1)
    q_acc[...] += lax.dot_general(oneh, eaone, (((0,), (0,)), ((), ())),
                                  preferred_element_type=jnp.float32)

    @pl.when(i == GRID - 1)
    def _():
        q = q_acc[...]
        inv = 1.0 / jnp.maximum(q[:, 4:5], 1.0)
        pooled = (jnp.dot(p_acc[...] * inv, w2x[...],
                          preferred_element_type=jnp.float32)
                  + jnp.dot(q[:, :4] * inv, w2e[...],
                            preferred_element_type=jnp.float32)
                  + b2[...])
        logits = jnp.dot(pooled, w3[...],
                         preferred_element_type=jnp.float32) + b3[...]
        mx = jnp.max(logits, axis=1, keepdims=True)
        lse = jnp.log(jnp.sum(jnp.exp(logits - mx), axis=1, keepdims=True))
        out[...] = logits - mx - lse


_tc_head = pl.pallas_call(
    _tc_body,
    grid=(GRID,),
    in_specs=[
        pl.BlockSpec((BLK, HD), lambda i: (i, 0)),
        pl.BlockSpec((BLK, HD), lambda i: (i + GRID, 0)),
        pl.BlockSpec((BLK, NG), lambda i: (i, 0)),
        pl.BlockSpec((BLK, 4), lambda i: (i, 0)),
        pl.BlockSpec((BLK, 1), lambda i: (i, 0)),
        pl.BlockSpec((HD, H), lambda i: (0, 0)),
        pl.BlockSpec((HD, H), lambda i: (0, 0)),
        pl.BlockSpec((4, H), lambda i: (0, 0)),
        pl.BlockSpec((1, H), lambda i: (0, 0)),
        pl.BlockSpec((H, H), lambda i: (0, 0)),
        pl.BlockSpec((4, H), lambda i: (0, 0)),
        pl.BlockSpec((1, H), lambda i: (0, 0)),
        pl.BlockSpec((H, OUT), lambda i: (0, 0)),
        pl.BlockSpec((1, OUT), lambda i: (0, 0)),
    ],
    out_specs=pl.BlockSpec((NG, OUT), lambda i: (0, 0)),
    out_shape=jax.ShapeDtypeStruct((NG, OUT), jnp.float32),
    scratch_shapes=[pltpu.VMEM((NG, H), jnp.float32),
                    pltpu.VMEM((NG, 8), jnp.float32)],
)


def kernel(x, edge_index, edge_attr, batch, W1, b1, W2, b2, W3, b3):
    zpad = jnp.zeros((NP - N, HD), jnp.float32)
    x2 = jnp.concatenate([x[:, :HD], zpad, x[:, HD:], zpad], axis=0)
    srcr = edge_index[0].reshape(NSUB, NCH, CH)
    dstr = edge_index[1].reshape(NSUB, NCH, CH)
    zc = jnp.zeros((CW,), jnp.float32)
    znp = jnp.zeros((NP,), jnp.float32)
    srcf = edge_index[0].reshape(NSUB, EPS)
    (agg,) = _sc_agg(x2, srcf, dstr)
    cflat, e0, e1, e2 = _sc_cnt(srcr, dstr, edge_attr[:, 0], edge_attr[:, 1],
                                edge_attr[:, 2], batch, zc, znp)
    cmat = cflat.reshape(NP, NG)
    eas = jnp.stack([e0, e1, e2, znp], axis=1)
    batch2 = jnp.pad(batch.reshape(N, 1), ((0, NP - N), (0, 0)),
                     constant_values=NG)
    w1a = W1[:, :HD].T
    w1b = W1[:, HD:DIN].T
    w1e = jnp.pad(W1[:, DIN:].T, ((0, 1), (0, 0)))
    b1r = b1.reshape(1, H)
    w2x = W2[:, :H].T
    w2e = jnp.pad(W2[:, H:].T, ((0, 1), (0, 0)))
    b2r = b2.reshape(1, H)
    w3t = W3.T
    b3r = b3.reshape(1, OUT)
    return _tc_head(agg, agg, cmat, eas, batch2, w1a, w1b, w1e, b1r,
                    w2x, w2e, b2r, w3t, b3r)


# speedup vs baseline: 23.7496x; 1.0224x over previous
import jax
import jax.numpy as jnp
from jax import lax
from jax.experimental import pallas as pl
from jax.experimental.pallas import tpu as pltpu
from jax.experimental.pallas import tpu_sc as plsc

N = 10000
NP = 10240
E = 160000
DIN = 256
HD = 128
H = 512
OUT = 40
NG = 64
NSUB = 16
EPS = E // NSUB
CH = 80
NCH = EPS // CH
RPS = NP // NSUB
CW = NP * NG
CPS = CW // NSUB
BLK = 2048
GRID = NP // BLK

_SC_PARAMS = pltpu.CompilerParams(needs_layout_passes=False)


def _sc_agg_body(x2_hbm, srcf_hbm, dst_hbm, agg_hbm,
                 src_v, dst_v, rows0_v, rows1_v, agg_sh,
                 gsem0, gsem1, ssem0, ssem1):
    c = lax.axis_index("c")
    s = lax.axis_index("s")

    pltpu.sync_copy(srcf_hbm.at[s], src_v)
    pltpu.sync_copy(dst_hbm.at[s], dst_v)
    pltpu.sync_copy(x2_hbm.at[pl.ds(c * NP + s * RPS, RPS), :],
                    agg_sh.at[pl.ds(s * RPS, RPS), :])

    @pl.when(c == 1)
    def _():
        def build(r, carry):
            sl = pl.ds(r * 16, 16)
            src_v[sl] = src_v[sl] + NP
            return carry
        lax.fori_loop(0, EPS // 16, build, 0)

    plsc.subcore_barrier()

    def gather(j, buf, sem):
        pltpu.async_copy(x2_hbm.at[src_v.at[pl.ds(j * CH, CH)]], buf, sem)

    def gwait(buf, sem):
        pltpu.make_async_copy(x2_hbm.at[pl.ds(0, CH), :], buf, sem).wait()

    def sfire(j, buf, sem):
        pltpu.async_copy(buf, agg_sh.at[dst_v.at[j]], sem, add=True)

    def swait(buf, sem):
        pltpu.make_async_copy(x2_hbm.at[pl.ds(0, CH), :], buf, sem).wait()

    gather(0, rows0_v, gsem0)
    gather(1, rows1_v, gsem1)

    def step(t, carry):
        j0 = 2 * t
        gwait(rows0_v, gsem0)
        sfire(j0, rows0_v, ssem0)
        gwait(rows1_v, gsem1)
        sfire(j0 + 1, rows1_v, ssem1)
        swait(rows0_v, ssem0)
        gather(j0 + 2, rows0_v, gsem0)
        swait(rows1_v, ssem1)

        @pl.when(j0 + 3 < NCH)
        def _():
            gather(j0 + 3, rows1_v, gsem1)
        return carry

    lax.fori_loop(0, (NCH - 1) // 2, step, 0)
    gwait(rows0_v, gsem0)
    sfire(NCH - 1, rows0_v, ssem0)
    swait(rows0_v, ssem0)
    plsc.subcore_barrier()

    pltpu.sync_copy(agg_sh.at[pl.ds(s * RPS, RPS), :],
                    agg_hbm.at[pl.ds(c * NP + s * RPS, RPS), :])


_sc_agg = pl.kernel(
    _sc_agg_body,
    mesh=plsc.VectorSubcoreMesh(core_axis_name="c", subcore_axis_name="s"),
    compiler_params=_SC_PARAMS,
    out_type=[jax.ShapeDtypeStruct((2 * NP, HD), jnp.float32)],
    scratch_types=[
        pltpu.VMEM((EPS,), jnp.int32),
        pltpu.VMEM((NCH, CH), jnp.int32),
        pltpu.VMEM((CH, HD), jnp.float32),
        pltpu.VMEM((CH, HD), jnp.float32),
        pltpu.VMEM_SHARED((NP, HD), jnp.float32),
        pltpu.SemaphoreType.DMA,
        pltpu.SemaphoreType.DMA,
        pltpu.SemaphoreType.DMA,
        pltpu.SemaphoreType.DMA,
    ],
)


def _sc_cnt_body(src_hbm, dst_hbm, ea0_hbm, ea1_hbm, ea2_hbm, batch_hbm,
                 znp_hbm,
                 cnt_hbm, e0_hbm, e1_hbm, e2_hbm,
                 src_v, dst_v, batch_v, ea0_v, ea1_v, ea2_v, ones_v, zb_v,
                 c_sh, e0_sh, e1_sh, e2_sh, ssem):
    c = lax.axis_index("c")
    s = lax.axis_index("s")

    pltpu.sync_copy(src_hbm.at[s], src_v)
    pltpu.sync_copy(dst_hbm.at[s], dst_v)

    @pl.when(c == 0)
    def _():
        pltpu.sync_copy(batch_hbm, batch_v)

        def zfill(k, carry):
            zb_v[pl.ds(k * 16, 16)] = jnp.zeros((16,), jnp.float32)
            return carry
        lax.fori_loop(0, 2560 // 16, zfill, 0)

        def zout(k, carry):
            pltpu.sync_copy(zb_v, c_sh.at[pl.ds(s * CPS + k * 2560, 2560)])
            return carry
        lax.fori_loop(0, CPS // 2560, zout, 0)
        for k in range(CH // 16):
            ones_v[pl.ds(k * 16, 16)] = jnp.ones((16,), jnp.float32)

    @pl.when(c == 1)
    def _():
        pltpu.sync_copy(ea0_hbm.at[pl.ds(s * EPS, EPS)], ea0_v)
        pltpu.sync_copy(ea1_hbm.at[pl.ds(s * EPS, EPS)], ea1_v)
        pltpu.sync_copy(ea2_hbm.at[pl.ds(s * EPS, EPS)], ea2_v)
        pltpu.sync_copy(znp_hbm.at[pl.ds(s * RPS, RPS)],
                        e0_sh.at[pl.ds(s * RPS, RPS)])
        pltpu.sync_copy(znp_hbm.at[pl.ds(s * RPS, RPS)],
                        e1_sh.at[pl.ds(s * RPS, RPS)])
        pltpu.sync_copy(znp_hbm.at[pl.ds(s * RPS, RPS)],
                        e2_sh.at[pl.ds(s * RPS, RPS)])

    @pl.when(c == 0)
    def _():
        def build(r, carry):
            for k in range(CH // 16):
                sl = pl.ds(k * 16, 16)
                d16 = dst_v[r, sl]
                g16 = plsc.load_gather(batch_v, [d16])
                src_v[r, sl] = src_v[r, sl] * NG + g16
            return carry
        lax.fori_loop(0, NCH, build, 0)

    plsc.subcore_barrier()

    def step(j, carry):
        @pl.when(c == 0)
        def _():
            pltpu.async_copy(ones_v, c_sh.at[src_v.at[j]], ssem, add=True)

        @pl.when(c == 1)
        def _():
            pltpu.async_copy(ea0_v.at[pl.ds(j * CH, CH)],
                             e0_sh.at[dst_v.at[j]], ssem, add=True)
            pltpu.async_copy(ea1_v.at[pl.ds(j * CH, CH)],
                             e1_sh.at[dst_v.at[j]], ssem, add=True)
            pltpu.async_copy(ea2_v.at[pl.ds(j * CH, CH)],
                             e2_sh.at[dst_v.at[j]], ssem, add=True)
        return carry

    lax.fori_loop(0, NCH, step, 0)

    def drain(j, carry):
        @pl.when(c == 0)
        def _():
            pltpu.make_async_copy(znp_hbm.at[pl.ds(0, CH)], ones_v, ssem).wait()

        @pl.when(c == 1)
        def _():
            pltpu.make_async_copy(znp_hbm.at[pl.ds(0, CH)],
                                  ea0_v.at[pl.ds(0, CH)], ssem).wait()
            pltpu.make_async_copy(znp_hbm.at[pl.ds(0, CH)],
                                  ea1_v.at[pl.ds(0, CH)], ssem).wait()
            pltpu.make_async_copy(znp_hbm.at[pl.ds(0, CH)],
                                  ea2_v.at[pl.ds(0, CH)], ssem).wait()
        return carry

    lax.fori_loop(0, NCH, drain, 0)
    plsc.subcore_barrier()

    @pl.when(c == 0)
    def _():
        pltpu.sync_copy(c_sh.at[pl.ds(s * CPS, CPS)],
                        cnt_hbm.at[pl.ds(s * CPS, CPS)])

    @pl.when(c == 1)
    def _():
        pltpu.sync_copy(e0_sh.at[pl.ds(s * RPS, RPS)],
                        e0_hbm.at[pl.ds(s * RPS, RPS)])
        pltpu.sync_copy(e1_sh.at[pl.ds(s * RPS, RPS)],
                        e1_hbm.at[pl.ds(s * RPS, RPS)])
        pltpu.sync_copy(e2_sh.at[pl.ds(s * RPS, RPS)],
                        e2_hbm.at[pl.ds(s * RPS, RPS)])


_sc_cnt = pl.kernel(
    _sc_cnt_body,
    mesh=plsc.VectorSubcoreMesh(core_axis_name="c", subcore_axis_name="s"),
    compiler_params=_SC_PARAMS,
    out_type=[jax.ShapeDtypeStruct((CW,), jnp.float32),
              jax.ShapeDtypeStruct((NP,), jnp.float32),
              jax.ShapeDtypeStruct((NP,), jnp.float32),
              jax.ShapeDtypeStruct((NP,), jnp.float32)],
    scratch_types=[
        pltpu.VMEM((NCH, CH), jnp.int32),
        pltpu.VMEM((NCH, CH), jnp.int32),
        pltpu.VMEM((N,), jnp.int32),
        pltpu.VMEM((EPS,), jnp.float32),
        pltpu.VMEM((EPS,), jnp.float32),
        pltpu.VMEM((EPS,), jnp.float32),
        pltpu.VMEM((CH,), jnp.float32),
        pltpu.VMEM((2560,), jnp.float32),
        pltpu.VMEM_SHARED((CW,), jnp.float32),
        pltpu.VMEM_SHARED((NP,), jnp.float32),
        pltpu.VMEM_SHARED((NP,), jnp.float32),
        pltpu.VMEM_SHARED((NP,), jnp.float32),
        pltpu.SemaphoreType.DMA,
    ],
)


def _tc_body(agg0, agg1, cmat, eas, batch, w1a, w1b, w1e, b1, w2x, w2e, b2,
             w3, b3, out, p_acc, q_acc):
    i = pl.program_id(0)

    @pl.when(i == 0)
    def _():
        p_acc[...] = jnp.zeros_like(p_acc)
        q_acc[...] = jnp.zeros_like(q_acc)

    h = (jnp.dot(agg0[...], w1a[...], preferred_element_type=jnp.float32)
         + jnp.dot(agg1[...], w1b[...], preferred_element_type=jnp.float32)
         + jnp.dot(eas[...], w1e[...], preferred_element_type=jnp.float32)
         + b1[...])
    h = jnp.maximum(h, 0.0)
    gids = lax.broadcasted_iota(jnp.int32, (1, NG), 1)
    oneh = (batch[...] == gids).astype(jnp.float32)
    m = oneh + cmat[...]
    p_acc[...] += lax.dot_general(m, h, (((0,), (0,)), ((), ())),
                                  preferred_element_type=jnp.float32)
    eaone = jnp.concatenate(
        [eas[...], jnp.ones((BLK, 1), jnp.float32),
         jnp.zeros((BLK, 3), jnp.float32)], axis=1)
    q_acc[...] += lax.dot_general(oneh, eaone, (((0,), (0,)), ((), ())),
                                  preferred_element_type=jnp.float32)

    @pl.when(i == GRID - 1)
    def _():
        q = q_acc[...]
        inv = 1.0 / jnp.maximum(q[:, 4:5], 1.0)
        pooled = (jnp.dot(p_acc[...] * inv, w2x[...],
                          preferred_element_type=jnp.float32)
                  + jnp.dot(q[:, :4] * inv, w2e[...],
                            preferred_element_type=jnp.float32)
                  + b2[...])
        logits = jnp.dot(pooled, w3[...],
                         preferred_element_type=jnp.float32) + b3[...]
        mx = jnp.max(logits, axis=1, keepdims=True)
        lse = jnp.log(jnp.sum(jnp.exp(logits - mx), axis=1, keepdims=True))
        out[...] = logits - mx - lse


_tc_head = pl.pallas_call(
    _tc_body,
    grid=(GRID,),
    in_specs=[
        pl.BlockSpec((BLK, HD), lambda i: (i, 0)),
        pl.BlockSpec((BLK, HD), lambda i: (i + GRID, 0)),
        pl.BlockSpec((BLK, NG), lambda i: (i, 0)),
        pl.BlockSpec((BLK, 4), lambda i: (i, 0)),
        pl.BlockSpec((BLK, 1), lambda i: (i, 0)),
        pl.BlockSpec((HD, H), lambda i: (0, 0)),
        pl.BlockSpec((HD, H), lambda i: (0, 0)),
        pl.BlockSpec((4, H), lambda i: (0, 0)),
        pl.BlockSpec((1, H), lambda i: (0, 0)),
        pl.BlockSpec((H, H), lambda i: (0, 0)),
        pl.BlockSpec((4, H), lambda i: (0, 0)),
        pl.BlockSpec((1, H), lambda i: (0, 0)),
        pl.BlockSpec((H, OUT), lambda i: (0, 0)),
        pl.BlockSpec((1, OUT), lambda i: (0, 0)),
    ],
    out_specs=pl.BlockSpec((NG, OUT), lambda i: (0, 0)),
    out_shape=jax.ShapeDtypeStruct((NG, OUT), jnp.float32),
    scratch_shapes=[pltpu.VMEM((NG, H), jnp.float32),
                    pltpu.VMEM((NG, 8), jnp.float32)],
)


def kernel(x, edge_index, edge_attr, batch, W1, b1, W2, b2, W3, b3):
    zpad = jnp.zeros((NP - N, HD), jnp.float32)
    x2 = jnp.concatenate([x[:, :HD], zpad, x[:, HD:], zpad], axis=0)
    srcr = edge_index[0].reshape(NSUB, NCH, CH)
    dstr = edge_index[1].reshape(NSUB, NCH, CH)
    znp = jnp.zeros((NP,), jnp.float32)
    srcf = edge_index[0].reshape(NSUB, EPS)
    (agg,) = _sc_agg(x2, srcf, dstr)
    cflat, e0, e1, e2 = _sc_cnt(srcr, dstr, edge_attr[:, 0], edge_attr[:, 1],
                                edge_attr[:, 2], batch, znp)
    cmat = cflat.reshape(NP, NG)
    eas = jnp.stack([e0, e1, e2, znp], axis=1)
    batch2 = jnp.pad(batch.reshape(N, 1), ((0, NP - N), (0, 0)),
                     constant_values=NG)
    w1a = W1[:, :HD].T
    w1b = W1[:, HD:DIN].T
    w1e = jnp.pad(W1[:, DIN:].T, ((0, 1), (0, 0)))
    b1r = b1.reshape(1, H)
    w2x = W2[:, :H].T
    w2e = jnp.pad(W2[:, H:].T, ((0, 1), (0, 0)))
    b2r = b2.reshape(1, H)
    w3t = W3.T
    b3r = b3.reshape(1, OUT)
    return _tc_head(agg, agg, cmat, eas, batch2, w1a, w1b, w1e, b1r,
                    w2x, w2e, b2r, w3t, b3r)
